# baseline (speedup 1.0000x reference)
import jax
import jax.numpy as jnp
from jax.experimental import pallas as pl

B = 4096
N = 50
D = 128
H1 = 256
OUT = 128
R = 3

BB = 64
T = BB * N


def _lk(x):
    return jnp.maximum(x, x * jnp.asarray(0.01, x.dtype))


def _fused_body(u_ref, i_ref, a_ref, o_ref, s_ref, e_ref,
                uw0u_ref, uw0i_ref, ub0_ref, uw1_ref, ub1_ref,
                aw0_ref, ab0_ref, aw1_ref, ab1_ref,
                pred_ref):
    f32 = jnp.float32
    bf16 = jnp.bfloat16

    u = u_ref[...].astype(bf16)
    i = i_ref[...].astype(bf16)
    h_ui = _lk(
        jnp.dot(u, uw0u_ref[...], preferred_element_type=f32)
        + jnp.dot(i, uw0i_ref[...], preferred_element_type=f32)
        + ub0_ref[...]
    )
    ue = _lk(
        jnp.dot(h_ui.astype(bf16), uw1_ref[...], preferred_element_type=f32)
        + ub1_ref[...]
    )

    ue_tok = jnp.dot(e_ref[...], ue.astype(bf16),
                     preferred_element_type=f32)

    xa = a_ref[...].astype(bf16).reshape(T, D)
    xo = o_ref[...].astype(bf16).reshape(T, D)
    x = jnp.concatenate([xa, xo], axis=1)

    h_all = jnp.dot(x, aw0_ref[...], preferred_element_type=f32) + ab0_ref[...]
    h_all = _lk(h_all.astype(bf16))

    s = s_ref[0, 0, :]
    pred = jnp.zeros((T,), dtype=f32)
    for r in range(R):
        h_r = h_all[:, r * H1:(r + 1) * H1]
        out_r = _lk(
            jnp.dot(h_r, aw1_ref[r], preferred_element_type=f32)
            + ab1_ref[r]
        )
        p_r = jnp.sum(out_r * ue_tok, axis=-1)
        pred = pred + jnp.where(s == r, p_r, 0.0)
    pred_ref[0, 0, :] = pred


def kernel(u_emb, i_emb, a_emb, o_emb, s, ui_W0, ui_b0, ui_W1, ui_b1,
           ao_W0, ao_b0, ao_W1, ao_b1):
    bf16 = jnp.bfloat16
    uw0u = ui_W0[:, :D].T.astype(bf16)
    uw0i = ui_W0[:, D:].T.astype(bf16)
    uw1 = ui_W1.T.astype(bf16)
    aw0 = ao_W0.transpose(2, 0, 1).reshape(2 * D, R * H1).astype(bf16)
    ab0 = ao_b0.reshape(R * H1)
    aw1 = ao_W1.transpose(0, 2, 1).astype(bf16)
    s_flat = s.astype(jnp.int32).reshape(B // BB, 1, T)
    e_mat = (jnp.arange(T, dtype=jnp.int32)[:, None] // N
             == jnp.arange(BB, dtype=jnp.int32)[None, :]).astype(bf16)

    grid = (B // BB,)

    def const(shape):
        nd = len(shape)
        return pl.BlockSpec(shape, lambda i: (0,) * nd)

    out = pl.pallas_call(
        _fused_body,
        grid=grid,
        in_specs=[
            pl.BlockSpec((BB, D), lambda i: (i, 0)),
            pl.BlockSpec((BB, D), lambda i: (i, 0)),
            pl.BlockSpec((BB, N, D), lambda i: (i, 0, 0)),
            pl.BlockSpec((BB, N, D), lambda i: (i, 0, 0)),
            pl.BlockSpec((1, 1, T), lambda i: (i, 0, 0)),
            const((T, BB)),
            const((D, H1)), const((D, H1)), const((H1,)),
            const((H1, OUT)), const((OUT,)),
            const((2 * D, R * H1)), const((R * H1,)),
            const((R, H1, OUT)), const((R, OUT)),
        ],
        out_specs=pl.BlockSpec((1, 1, T), lambda i: (i, 0, 0)),
        out_shape=jax.ShapeDtypeStruct((B // BB, 1, T), jnp.float32),
    )(u_emb, i_emb, a_emb, o_emb, s_flat, e_mat,
      uw0u, uw0i, ui_b0, uw1, ui_b1,
      aw0, ab0, aw1, ao_b1)
    return out.reshape(B, N)


# speedup vs baseline: 2.5520x; 2.5520x over previous
import jax
import jax.numpy as jnp
from jax.experimental import pallas as pl

B = 4096
N = 50
D = 128
H1 = 256
OUT = 128
R = 3

BB = 64
T = BB * N


def _lk(x):
    return jnp.maximum(x, x * jnp.asarray(0.01, x.dtype))


def _fused_body(u_ref, i_ref, a_ref, o_ref, s_ref,
                uw0u_ref, uw0i_ref, ub0_ref, uw1_ref, ub1_ref,
                aw0_ref, ab0_ref, aw1_ref, ab1_ref,
                pred_ref):
    f32 = jnp.float32
    bf16 = jnp.bfloat16

    u = u_ref[...].astype(bf16)
    i = i_ref[...].astype(bf16)
    h_ui = _lk(
        jnp.dot(u, uw0u_ref[...], preferred_element_type=f32)
        + jnp.dot(i, uw0i_ref[...], preferred_element_type=f32)
        + ub0_ref[...]
    )
    ue = _lk(
        jnp.dot(h_ui.astype(bf16), uw1_ref[...], preferred_element_type=f32)
        + ub1_ref[...]
    )

    xa = a_ref[...].astype(bf16).reshape(T, D)
    xo = o_ref[...].astype(bf16).reshape(T, D)
    x = jnp.concatenate([xa, xo], axis=1)

    h_all = jnp.dot(x, aw0_ref[...], preferred_element_type=f32) + ab0_ref[...]
    h_all = _lk(h_all.astype(bf16))

    s = s_ref[...]
    ue_b = ue[:, None, :]
    pred = jnp.zeros((BB, N), dtype=f32)
    for r in range(R):
        h_r = h_all[:, r * H1:(r + 1) * H1]
        out_r = _lk(
            jnp.dot(h_r, aw1_ref[r], preferred_element_type=f32)
            + ab1_ref[r]
        )
        p_r = jnp.sum(out_r.reshape(BB, N, OUT) * ue_b, axis=-1)
        pred = pred + jnp.where(s == r, p_r, 0.0)
    pred_ref[...] = pred


def kernel(u_emb, i_emb, a_emb, o_emb, s, ui_W0, ui_b0, ui_W1, ui_b1,
           ao_W0, ao_b0, ao_W1, ao_b1):
    bf16 = jnp.bfloat16
    uw0u = ui_W0[:, :D].T.astype(bf16)
    uw0i = ui_W0[:, D:].T.astype(bf16)
    uw1 = ui_W1.T.astype(bf16)
    aw0 = ao_W0.transpose(2, 0, 1).reshape(2 * D, R * H1).astype(bf16)
    ab0 = ao_b0.reshape(R * H1)
    aw1 = ao_W1.transpose(0, 2, 1).astype(bf16)
    s32 = s.astype(jnp.int32)

    grid = (B // BB,)

    def const(shape):
        nd = len(shape)
        return pl.BlockSpec(shape, lambda i: (0,) * nd)

    out = pl.pallas_call(
        _fused_body,
        grid=grid,
        in_specs=[
            pl.BlockSpec((BB, D), lambda i: (i, 0)),
            pl.BlockSpec((BB, D), lambda i: (i, 0)),
            pl.BlockSpec((BB, N, D), lambda i: (i, 0, 0)),
            pl.BlockSpec((BB, N, D), lambda i: (i, 0, 0)),
            pl.BlockSpec((BB, N), lambda i: (i, 0)),
            const((D, H1)), const((D, H1)), const((H1,)),
            const((H1, OUT)), const((OUT,)),
            const((2 * D, R * H1)), const((R * H1,)),
            const((R, H1, OUT)), const((R, OUT)),
        ],
        out_specs=pl.BlockSpec((BB, N), lambda i: (i, 0)),
        out_shape=jax.ShapeDtypeStruct((B, N), jnp.float32),
    )(u_emb, i_emb, a_emb, o_emb, s32,
      uw0u, uw0i, ui_b0, uw1, ui_b1,
      aw0, ab0, aw1, ao_b1)
    return out


# speedup vs baseline: 3.5798x; 1.4027x over previous
import jax
import jax.numpy as jnp
from jax.experimental import pallas as pl

B = 4096
N = 50
D = 128
H1 = 256
OUT = 128
R = 3

BB = 64
NP = 56
T = BB * NP


def _lk(x):
    return jnp.maximum(x, x * jnp.asarray(0.01, x.dtype))


def _fused_body(u_ref, i_ref, a_ref, o_ref, s_ref,
                uw0u_ref, uw0i_ref, ub0_ref, uw1_ref, ub1_ref,
                aw0_ref, ab0_ref, aw1_ref, ab1_ref,
                pred_ref):
    f32 = jnp.float32
    bf16 = jnp.bfloat16

    u = u_ref[...].astype(bf16)
    i = i_ref[...].astype(bf16)
    h_ui = _lk(
        jnp.dot(u, uw0u_ref[...], preferred_element_type=f32)
        + jnp.dot(i, uw0i_ref[...], preferred_element_type=f32)
        + ub0_ref[...]
    )
    ue = _lk(
        jnp.dot(h_ui.astype(bf16), uw1_ref[...], preferred_element_type=f32)
        + ub1_ref[...]
    )

    zpad = jnp.zeros((BB, NP - N, D), dtype=f32)
    xa = jnp.concatenate([a_ref[...], zpad], axis=1).reshape(T, D).astype(bf16)
    xo = jnp.concatenate([o_ref[...], zpad], axis=1).reshape(T, D).astype(bf16)
    x = jnp.concatenate([xa, xo], axis=1)

    h_all = jnp.dot(x, aw0_ref[...], preferred_element_type=f32) + ab0_ref[...]
    h_all = _lk(h_all.astype(bf16))

    s = s_ref[...]
    ue_b = ue[:, None, :]
    pred = jnp.zeros((BB, N), dtype=f32)
    for r in range(R):
        h_r = h_all[:, r * H1:(r + 1) * H1]
        out_r = _lk(
            jnp.dot(h_r, aw1_ref[r], preferred_element_type=f32)
            + ab1_ref[r]
        )
        p_r = jnp.sum(out_r.reshape(BB, NP, OUT) * ue_b, axis=-1)
        pred = pred + jnp.where(s == r, p_r[:, :N], 0.0)
    pred_ref[...] = pred


def kernel(u_emb, i_emb, a_emb, o_emb, s, ui_W0, ui_b0, ui_W1, ui_b1,
           ao_W0, ao_b0, ao_W1, ao_b1):
    bf16 = jnp.bfloat16
    uw0u = ui_W0[:, :D].T.astype(bf16)
    uw0i = ui_W0[:, D:].T.astype(bf16)
    uw1 = ui_W1.T.astype(bf16)
    aw0 = ao_W0.transpose(2, 0, 1).reshape(2 * D, R * H1).astype(bf16)
    ab0 = ao_b0.reshape(R * H1)
    aw1 = ao_W1.transpose(0, 2, 1).astype(bf16)
    s32 = s.astype(jnp.int32)

    grid = (B // BB,)

    def const(shape):
        nd = len(shape)
        return pl.BlockSpec(shape, lambda i: (0,) * nd)

    out = pl.pallas_call(
        _fused_body,
        grid=grid,
        in_specs=[
            pl.BlockSpec((BB, D), lambda i: (i, 0)),
            pl.BlockSpec((BB, D), lambda i: (i, 0)),
            pl.BlockSpec((BB, N, D), lambda i: (i, 0, 0)),
            pl.BlockSpec((BB, N, D), lambda i: (i, 0, 0)),
            pl.BlockSpec((BB, N), lambda i: (i, 0)),
            const((D, H1)), const((D, H1)), const((H1,)),
            const((H1, OUT)), const((OUT,)),
            const((2 * D, R * H1)), const((R * H1,)),
            const((R, H1, OUT)), const((R, OUT)),
        ],
        out_specs=pl.BlockSpec((BB, N), lambda i: (i, 0)),
        out_shape=jax.ShapeDtypeStruct((B, N), jnp.float32),
    )(u_emb, i_emb, a_emb, o_emb, s32,
      uw0u, uw0i, ui_b0, uw1, ui_b1,
      aw0, ab0, aw1, ao_b1)
    return out
